# baseline (speedup 1.0000x reference)
import functools

import jax
import jax.numpy as jnp
from jax import lax
from jax.experimental import pallas as pl
from jax.experimental.pallas import tpu as pltpu
from jax.experimental.pallas import tpu_sc as plsc

NUM_EMBEDDINGS = 100000
EMBEDDING_DIM = 128

_INFO = plsc.get_sparse_core_info()
_NC = _INFO.num_cores
_NS = _INFO.num_subcores
_NW = _NC * _NS

_IDX_COLS = 128
_RPC = 2
_D = 3
_G = 1


def _gather_body(weight_hbm, idx_hbm, out_hbm, idx_v, rows_v,
                 gsems, osems, *, rows_per_worker):
    wid = lax.axis_index("s") * _NC + lax.axis_index("c")
    row_base = wid * rows_per_worker
    n_chunks = rows_per_worker // _RPC
    n_outer = (n_chunks + _D - 1) // _D

    pltpu.sync_copy(idx_hbm.at[pl.ds(row_base, rows_per_worker)], idx_v)

    def gather_copies(c, b):
        return [
            pltpu.make_async_copy(
                weight_hbm.at[idx_v.at[c * _RPC + r]],
                rows_v.at[b, pl.ds(r * _IDX_COLS, _IDX_COLS)],
                gsems[b],
            )
            for r in range(_RPC)
        ]

    def out_copy(c, b):
        return pltpu.make_async_copy(
            rows_v.at[b],
            out_hbm.at[pl.ds((row_base + c * _RPC) * _IDX_COLS,
                             _RPC * _IDX_COLS)],
            osems[b],
        )

    for b in range(_G):
        for cp in gather_copies(b, b):
            cp.start()

    def outer(i):
        for b in range(_D):
            c = i * _D + b
            nb = (b + _G) % _D

            @pl.when(c + _G < n_chunks)
            def _():
                @pl.when(c + _G >= _D)
                def _():
                    out_copy(0, nb).wait()
                for cp in gather_copies(c + _G, nb):
                    cp.start()

            @pl.when(c < n_chunks)
            def _():
                for cp in gather_copies(c, b):
                    cp.wait()
                out_copy(c, b).start()

    pl.loop(0, n_outer)(outer)

    for b in range(_D):
        out_copy(0, b).wait()


def kernel(x, weight):
    b, t = x.shape
    n_idx = b * t
    assert n_idx % (_NW * _IDX_COLS * _RPC) == 0
    idx_rows = n_idx // _IDX_COLS
    rows_per_worker = idx_rows // _NW

    xf = x.reshape(idx_rows, _IDX_COLS).astype(jnp.int32)

    mesh = plsc.VectorSubcoreMesh(core_axis_name="c", subcore_axis_name="s")
    body = functools.partial(_gather_body, rows_per_worker=rows_per_worker)
    out = pl.kernel(
        body,
        mesh=mesh,
        out_type=jax.ShapeDtypeStruct((n_idx, EMBEDDING_DIM), jnp.float32),
        scratch_types=[
            pltpu.VMEM((rows_per_worker, _IDX_COLS), jnp.int32),
            pltpu.VMEM((_D, _RPC * _IDX_COLS, EMBEDDING_DIM), jnp.float32),
            [pltpu.SemaphoreType.DMA] * _D,
            [pltpu.SemaphoreType.DMA] * _D,
        ],
    )(weight, xf)
    return out.reshape(b, t, EMBEDDING_DIM)


# speedup vs baseline: 1.0027x; 1.0027x over previous
import functools

import jax
import jax.numpy as jnp
from jax import lax
from jax.experimental import pallas as pl
from jax.experimental.pallas import tpu as pltpu
from jax.experimental.pallas import tpu_sc as plsc

NUM_EMBEDDINGS = 100000
EMBEDDING_DIM = 128

_INFO = plsc.get_sparse_core_info()
_NC = _INFO.num_cores
_NS = _INFO.num_subcores
_NW = _NC * _NS

_IDX_COLS = 128
_RPC = 1
_D = 6
_G = 3


def _gather_body(weight_hbm, idx_hbm, out_hbm, idx_v, rows_v,
                 gsems, osems, *, rows_per_worker):
    wid = lax.axis_index("s") * _NC + lax.axis_index("c")
    row_base = wid * rows_per_worker
    n_chunks = rows_per_worker // _RPC
    n_outer = (n_chunks + _D - 1) // _D

    pltpu.sync_copy(idx_hbm.at[pl.ds(row_base, rows_per_worker)], idx_v)

    def gather_copies(c, b):
        return [
            pltpu.make_async_copy(
                weight_hbm.at[idx_v.at[c * _RPC + r]],
                rows_v.at[b, pl.ds(r * _IDX_COLS, _IDX_COLS)],
                gsems[b],
            )
            for r in range(_RPC)
        ]

    def out_copy(c, b):
        return pltpu.make_async_copy(
            rows_v.at[b],
            out_hbm.at[pl.ds((row_base + c * _RPC) * _IDX_COLS,
                             _RPC * _IDX_COLS)],
            osems[b],
        )

    for b in range(_G):
        for cp in gather_copies(b, b):
            cp.start()

    def outer(i):
        for b in range(_D):
            c = i * _D + b
            nb = (b + _G) % _D

            @pl.when(c + _G < n_chunks)
            def _():
                @pl.when(c + _G >= _D)
                def _():
                    out_copy(0, nb).wait()
                for cp in gather_copies(c + _G, nb):
                    cp.start()

            @pl.when(c < n_chunks)
            def _():
                for cp in gather_copies(c, b):
                    cp.wait()
                out_copy(c, b).start()

    pl.loop(0, n_outer)(outer)

    for b in range(_D):
        out_copy(0, b).wait()


def kernel(x, weight):
    b, t = x.shape
    n_idx = b * t
    assert n_idx % (_NW * _IDX_COLS * _RPC) == 0
    idx_rows = n_idx // _IDX_COLS
    rows_per_worker = idx_rows // _NW

    xf = x.reshape(idx_rows, _IDX_COLS).astype(jnp.int32)

    mesh = plsc.VectorSubcoreMesh(core_axis_name="c", subcore_axis_name="s")
    body = functools.partial(_gather_body, rows_per_worker=rows_per_worker)
    out = pl.kernel(
        body,
        mesh=mesh,
        out_type=jax.ShapeDtypeStruct((n_idx, EMBEDDING_DIM), jnp.float32),
        scratch_types=[
            pltpu.VMEM((rows_per_worker, _IDX_COLS), jnp.int32),
            pltpu.VMEM((_D, _RPC * _IDX_COLS, EMBEDDING_DIM), jnp.float32),
            [pltpu.SemaphoreType.DMA] * _D,
            [pltpu.SemaphoreType.DMA] * _D,
        ],
    )(weight, xf)
    return out.reshape(b, t, EMBEDDING_DIM)
